# Initial kernel scaffold; baseline (speedup 1.0000x reference)
import jax
import jax.numpy as jnp
from jax.experimental import pallas as pl


def kernel(img, fV_region, seg):
    raise NotImplementedError("write your pallas kernel here")


# speedup vs baseline: 8.6601x; 8.6601x over previous
import functools

import jax
import jax.numpy as jnp
from jax import lax
from jax.experimental import pallas as pl
from jax.experimental.pallas import tpu as pltpu
from jax.experimental.pallas import tpu_sc as plsc

NC, NS, L = 2, 16, 16
NW = NC * NS


def _sums_kernel_body(nv, hw, px_per_tile, blk, img_ref, seg_ref, tabs_ref,
                      acc, segb, xb, shared, part, tbuf):
    c_id = lax.axis_index("c")
    s_id = lax.axis_index("s")
    wid = c_id * NS + s_id
    tw = 4 * nv

    zeros = jnp.zeros((L,), jnp.float32)

    def zbody(i, _):
        acc[pl.ds(i * L, L)] = zeros
        return _
    lax.fori_loop(0, tw // L, zbody, None)

    ones = jnp.full((L,), 1.0, jnp.float32)
    p0 = wid * px_per_tile
    b = p0 // hw
    i_base = p0 % hw
    nblk = px_per_tile // blk

    def block(bi, _):
        off = bi * blk
        pltpu.sync_copy(seg_ref.at[pl.ds(p0 + off, blk)], segb)
        for c in range(3):
            pltpu.sync_copy(
                img_ref.at[pl.ds((b * 3 + c) * hw + i_base + off, blk)],
                xb.at[pl.ds(c * blk, blk)])

        def vec(v, _):
            sv = segb[pl.ds(v * L, L)]
            for c in range(3):
                xv = xb[pl.ds(c * blk + v * L, L)]
                plsc.addupdate_scatter(acc, [sv + c * nv], xv)
            plsc.addupdate_scatter(acc, [sv + 3 * nv], ones)
            return _
        lax.fori_loop(0, blk // L, vec, None)
        return _
    lax.fori_loop(0, nblk, block, None)

    pltpu.sync_copy(acc, shared.at[pl.ds(s_id * tw, tw)])
    plsc.subcore_barrier()

    sl = tw // NS
    base = s_id * sl
    pltpu.sync_copy(shared.at[pl.ds(base, sl)], part)
    for t in range(1, NS):
        pltpu.sync_copy(shared.at[pl.ds(t * tw + base, sl)], tbuf)

        def radd(i, _):
            part[pl.ds(i * L, L)] = (part[pl.ds(i * L, L)]
                                     + tbuf[pl.ds(i * L, L)])
            return _
        lax.fori_loop(0, sl // L, radd, None)
    pltpu.sync_copy(part, tabs_ref.at[pl.ds(c_id * tw + base, sl)])


def _apply_kernel_body(nv, hw, px_per_tile, blk, img_ref, seg_ref, tabs_ref,
                       fvr_ref, out_ref, repl, t1, fvr, segb, xb, outb):
    c_id = lax.axis_index("c")
    s_id = lax.axis_index("s")
    wid = c_id * NS + s_id
    tw = 4 * nv

    pltpu.sync_copy(tabs_ref.at[pl.ds(0, tw)], repl)
    pltpu.sync_copy(tabs_ref.at[pl.ds(tw, tw)], t1)
    pltpu.sync_copy(fvr_ref, fvr)

    def tadd(i, _):
        repl[pl.ds(i * L, L)] = repl[pl.ds(i * L, L)] + t1[pl.ds(i * L, L)]
        return _
    lax.fori_loop(0, tw // L, tadd, None)

    one = jnp.full((L,), 1.0, jnp.float32)

    def rbody(i, _):
        cnt = repl[pl.ds(3 * nv + i * L, L)]
        cntm = jnp.maximum(cnt, one)
        for c in range(3):
            s = repl[pl.ds(c * nv + i * L, L)]
            f = fvr[pl.ds(c * nv + i * L, L)]
            repl[pl.ds(c * nv + i * L, L)] = f - s / cntm
        return _
    lax.fori_loop(0, nv // L, rbody, None)

    p0 = wid * px_per_tile
    b = p0 // hw
    i_base = p0 % hw
    nblk = px_per_tile // blk
    iota3 = lax.iota(jnp.int32, L) * 3

    def block(bi, _):
        off = bi * blk
        pltpu.sync_copy(seg_ref.at[pl.ds(p0 + off, blk)], segb)
        for c in range(3):
            pltpu.sync_copy(
                img_ref.at[pl.ds((b * 3 + c) * hw + i_base + off, blk)],
                xb.at[pl.ds(c * blk, blk)])

        def vec(v, _):
            sv = segb[pl.ds(v * L, L)]
            pos = iota3 + v * (3 * L)
            for c in range(3):
                xv = xb[pl.ds(c * blk + v * L, L)]
                rv = plsc.load_gather(repl, [sv + c * nv])
                plsc.store_scatter(outb, [pos + c], xv + rv)
            return _
        lax.fori_loop(0, blk // L, vec, None)
        pltpu.sync_copy(outb, out_ref.at[pl.ds((p0 + off) * 3, blk * 3)])
        return _
    lax.fori_loop(0, nblk, block, None)


def kernel(img, fV_region, seg):
    B, C, H, W = img.shape
    nv = fV_region.shape[0]
    n = B * H * W
    hw = H * W
    px_per_tile = n // NW
    tw = 4 * nv

    img_flat = img.reshape(-1)
    fvr_t = jnp.transpose(fV_region).reshape(-1)

    mesh = plsc.VectorSubcoreMesh(core_axis_name="c", subcore_axis_name="s")
    cparams = pltpu.CompilerParams(needs_layout_passes=False)

    blk_a = 8192
    sums = pl.kernel(
        functools.partial(_sums_kernel_body, nv, hw, px_per_tile, blk_a),
        out_type=jax.ShapeDtypeStruct((NC * tw,), jnp.float32),
        mesh=mesh,
        compiler_params=cparams,
        scratch_types=[
            pltpu.VMEM((tw,), jnp.float32),
            pltpu.VMEM((blk_a,), jnp.int32),
            pltpu.VMEM((3 * blk_a,), jnp.float32),
            pltpu.VMEM_SHARED((NS * tw,), jnp.float32),
            pltpu.VMEM((tw // NS,), jnp.float32),
            pltpu.VMEM((tw // NS,), jnp.float32),
        ],
    )(img_flat, seg)

    blk_c = 4096
    out = pl.kernel(
        functools.partial(_apply_kernel_body, nv, hw, px_per_tile, blk_c),
        out_type=jax.ShapeDtypeStruct((n * 3,), jnp.float32),
        mesh=mesh,
        compiler_params=cparams,
        scratch_types=[
            pltpu.VMEM((tw,), jnp.float32),
            pltpu.VMEM((tw,), jnp.float32),
            pltpu.VMEM((3 * nv,), jnp.float32),
            pltpu.VMEM((blk_c,), jnp.int32),
            pltpu.VMEM((3 * blk_c,), jnp.float32),
            pltpu.VMEM((3 * blk_c,), jnp.float32),
        ],
    )(img_flat, seg, sums, fvr_t)
    return out.reshape(n, 3)


# speedup vs baseline: 17.8339x; 2.0593x over previous
import functools

import jax
import jax.numpy as jnp
from jax import lax
from jax.experimental import pallas as pl
from jax.experimental.pallas import tpu as pltpu
from jax.experimental.pallas import tpu_sc as plsc

NC, NS, L = 2, 16, 16
NW = NC * NS

_GDN = lax.GatherDimensionNumbers(
    offset_dims=(), collapsed_slice_dims=(0,), start_index_map=(0,))


def _vgather(v, idx):
    return lax.gather(v, idx[:, None], _GDN, (1,),
                      mode=lax.GatherScatterMode.PROMISE_IN_BOUNDS)


def _sums_kernel_body(nv, hw, px_per_tile, blk, img_ref, seg_ref, tabs_ref,
                      acc, segb, xb, shared, part, tbuf):
    c_id = lax.axis_index("c")
    s_id = lax.axis_index("s")
    wid = c_id * NS + s_id
    tw = 4 * nv

    zeros = jnp.zeros((L,), jnp.float32)

    def zbody(i, _):
        acc[pl.ds(i * L, L)] = zeros
        return _
    lax.fori_loop(0, tw // L, zbody, None)

    iota = lax.iota(jnp.int32, L)
    i_up = jnp.minimum(iota + 1, L - 1)
    i_dn = jnp.maximum(iota - 1, 0)
    p0 = wid * px_per_tile
    b = p0 // hw
    i_base = p0 % hw
    nblk = px_per_tile // blk

    def block(bi, _):
        off = bi * blk
        pltpu.sync_copy(seg_ref.at[pl.ds(p0 + off, blk)], segb)
        for c in range(3):
            pltpu.sync_copy(
                img_ref.at[pl.ds((b * 3 + c) * hw + i_base + off, blk)],
                xb.at[pl.ds(c * blk, blk)])

        def vec(v, _):
            sv = segb[pl.ds(v * L, L)]
            is_first = jnp.logical_or(iota == 0, sv != _vgather(sv, i_dn))
            is_last = jnp.logical_or(iota == L - 1, sv != _vgather(sv, i_up))
            fol = plsc.cummax(jnp.where(is_first, iota, 0))
            for c in range(3):
                xv = xb[pl.ds(c * blk + v * L, L)]
                incl = plsc.cumsum(xv)
                base = _vgather(incl - xv, fol)
                plsc.addupdate_scatter(acc, [sv + c * nv], incl - base,
                                       mask=is_last)
            cntf = (iota - fol + 1).astype(jnp.float32)
            plsc.addupdate_scatter(acc, [sv + 3 * nv], cntf, mask=is_last)
            return _
        lax.fori_loop(0, blk // L, vec, None)
        return _
    lax.fori_loop(0, nblk, block, None)

    pltpu.sync_copy(acc, shared.at[pl.ds(s_id * tw, tw)])
    plsc.subcore_barrier()

    sl = tw // NS
    base = s_id * sl
    pltpu.sync_copy(shared.at[pl.ds(base, sl)], part)
    for t in range(1, NS):
        pltpu.sync_copy(shared.at[pl.ds(t * tw + base, sl)], tbuf)

        def radd(i, _):
            part[pl.ds(i * L, L)] = (part[pl.ds(i * L, L)]
                                     + tbuf[pl.ds(i * L, L)])
            return _
        lax.fori_loop(0, sl // L, radd, None)
    pltpu.sync_copy(part, tabs_ref.at[pl.ds(c_id * tw + base, sl)])


def _apply_kernel_body(nv, hw, px_per_tile, blk, n, img_ref, seg_ref,
                       tabs_ref, fvr_ref, out_ref, repl, t1, fvr, segb, xb,
                       outb):
    c_id = lax.axis_index("c")
    s_id = lax.axis_index("s")
    wid = c_id * NS + s_id
    tw = 4 * nv

    pltpu.sync_copy(tabs_ref.at[pl.ds(0, tw)], repl)
    pltpu.sync_copy(tabs_ref.at[pl.ds(tw, tw)], t1)
    pltpu.sync_copy(fvr_ref, fvr)

    def tadd(i, _):
        repl[pl.ds(i * L, L)] = repl[pl.ds(i * L, L)] + t1[pl.ds(i * L, L)]
        return _
    lax.fori_loop(0, tw // L, tadd, None)

    one = jnp.full((L,), 1.0, jnp.float32)

    def rbody(i, _):
        cnt = repl[pl.ds(3 * nv + i * L, L)]
        cntm = jnp.maximum(cnt, one)
        for c in range(3):
            s = repl[pl.ds(c * nv + i * L, L)]
            f = fvr[pl.ds(c * nv + i * L, L)]
            repl[pl.ds(c * nv + i * L, L)] = f - s / cntm
        return _
    lax.fori_loop(0, nv // L, rbody, None)

    p0 = wid * px_per_tile
    b = p0 // hw
    i_base = p0 % hw
    nblk = px_per_tile // blk

    def block(bi, _):
        off = bi * blk
        pltpu.sync_copy(seg_ref.at[pl.ds(p0 + off, blk)], segb)
        for c in range(3):
            pltpu.sync_copy(
                img_ref.at[pl.ds((b * 3 + c) * hw + i_base + off, blk)],
                xb.at[pl.ds(c * blk, blk)])

        def vec(v, _):
            sv = segb[pl.ds(v * L, L)]
            for c in range(3):
                xv = xb[pl.ds(c * blk + v * L, L)]
                rv = plsc.load_gather(repl, [sv + c * nv])
                outb[pl.ds(c * blk + v * L, L)] = xv + rv
            return _
        lax.fori_loop(0, blk // L, vec, None)
        for c in range(3):
            pltpu.sync_copy(outb.at[pl.ds(c * blk, blk)],
                            out_ref.at[pl.ds(c * n + p0 + off, blk)])
        return _
    lax.fori_loop(0, nblk, block, None)


def kernel(img, fV_region, seg):
    B, C, H, W = img.shape
    nv = fV_region.shape[0]
    n = B * H * W
    hw = H * W
    px_per_tile = n // NW
    tw = 4 * nv

    img_flat = img.reshape(-1)
    fvr_t = jnp.transpose(fV_region).reshape(-1)

    mesh = plsc.VectorSubcoreMesh(core_axis_name="c", subcore_axis_name="s")
    cparams = pltpu.CompilerParams(needs_layout_passes=False)

    blk_a = 8192
    sums = pl.kernel(
        functools.partial(_sums_kernel_body, nv, hw, px_per_tile, blk_a),
        out_type=jax.ShapeDtypeStruct((NC * tw,), jnp.float32),
        mesh=mesh,
        compiler_params=cparams,
        scratch_types=[
            pltpu.VMEM((tw,), jnp.float32),
            pltpu.VMEM((blk_a,), jnp.int32),
            pltpu.VMEM((3 * blk_a,), jnp.float32),
            pltpu.VMEM_SHARED((NS * tw,), jnp.float32),
            pltpu.VMEM((tw // NS,), jnp.float32),
            pltpu.VMEM((tw // NS,), jnp.float32),
        ],
    )(img_flat, seg)

    blk_c = 4096
    out = pl.kernel(
        functools.partial(_apply_kernel_body, nv, hw, px_per_tile, blk_c, n),
        out_type=jax.ShapeDtypeStruct((3 * n,), jnp.float32),
        mesh=mesh,
        compiler_params=cparams,
        scratch_types=[
            pltpu.VMEM((tw,), jnp.float32),
            pltpu.VMEM((tw,), jnp.float32),
            pltpu.VMEM((3 * nv,), jnp.float32),
            pltpu.VMEM((blk_c,), jnp.int32),
            pltpu.VMEM((3 * blk_c,), jnp.float32),
            pltpu.VMEM((3 * blk_c,), jnp.float32),
        ],
    )(img_flat, seg, sums, fvr_t)
    return jnp.transpose(out.reshape(3, n))


# speedup vs baseline: 47.3294x; 2.6539x over previous
import functools

import jax
import jax.numpy as jnp
from jax import lax
from jax.experimental import pallas as pl
from jax.experimental.pallas import tpu as pltpu
from jax.experimental.pallas import tpu_sc as plsc

NC, NS, L = 2, 16, 16
NW = NC * NS

_GDN = lax.GatherDimensionNumbers(
    offset_dims=(), collapsed_slice_dims=(0,), start_index_map=(0,))


def _vgather(v, idx):
    return lax.gather(v, idx[:, None], _GDN, (1,),
                      mode=lax.GatherScatterMode.PROMISE_IN_BOUNDS)


def _sums_kernel_body(nv, hw, px_per_tile, blk, img_ref, seg_ref, tabs_ref,
                      acc, segb, xb, shared, part, tbuf, sem_a, sem_b):
    c_id = lax.axis_index("c")
    s_id = lax.axis_index("s")
    wid = c_id * NS + s_id
    tw = 4 * nv

    zeros = jnp.zeros((L,), jnp.float32)

    def zbody(i, _):
        for u in range(4):
            acc[pl.ds((4 * i + u) * L, L)] = zeros
        return _
    lax.fori_loop(0, tw // (4 * L), zbody, None)

    iota = lax.iota(jnp.int32, L)
    i_up = jnp.minimum(iota + 1, L - 1)
    i_dn = jnp.maximum(iota - 1, 0)
    p0 = wid * px_per_tile
    b = p0 // hw
    i_base = p0 % hw
    nblk = px_per_tile // blk
    sems = (sem_a, sem_b)

    descs = {}

    def issue(bi, slot):
        off = bi * blk
        dd = [pltpu.async_copy(seg_ref.at[pl.ds(p0 + off, blk)],
                               segb.at[pl.ds(slot * blk, blk)], sems[slot])]
        for c in range(3):
            dd.append(pltpu.async_copy(
                img_ref.at[pl.ds((b * 3 + c) * hw + i_base + off, blk)],
                xb.at[pl.ds((slot * 3 + c) * blk, blk)], sems[slot]))
        descs[slot] = dd

    issue(0, 0)
    for bi in range(nblk):
        slot = bi & 1
        if bi + 1 < nblk:
            issue(bi + 1, 1 - slot)
        for d in descs[slot]:
            d.wait()

        def pvec(v):
            j0 = v * L
            poff = ((j0 >> 7) & 7) * 512 + (j0 >> 10) * 128 + (j0 & 127)
            sv = segb[pl.ds(slot * blk + poff, L)]
            is_first = jnp.logical_or(iota == 0, sv != _vgather(sv, i_dn))
            is_last = jnp.logical_or(iota == L - 1, sv != _vgather(sv, i_up))
            fol = plsc.cummax(jnp.where(is_first, iota, 0))
            for c in range(3):
                xv = xb[pl.ds((slot * 3 + c) * blk + v * L, L)]
                incl = plsc.cumsum(xv)
                base = _vgather(incl - xv, fol)
                plsc.addupdate_scatter(acc, [sv + c * nv], incl - base,
                                       mask=is_last)
            cntf = (iota - fol + 1).astype(jnp.float32)
            plsc.addupdate_scatter(acc, [sv + 3 * nv], cntf, mask=is_last)

        def vec(i, _):
            pvec(2 * i)
            pvec(2 * i + 1)
            return _
        lax.fori_loop(0, blk // (2 * L), vec, None)

    pltpu.sync_copy(acc, shared.at[pl.ds(s_id * tw, tw)])
    plsc.subcore_barrier()

    sl = tw // NS
    base = s_id * sl
    pltpu.sync_copy(shared.at[pl.ds(base, sl)], part)
    tds = {}

    def issue_t(t, slot):
        tds[slot] = pltpu.async_copy(shared.at[pl.ds(t * tw + base, sl)],
                                     tbuf.at[pl.ds(slot * sl, sl)],
                                     sems[slot])

    issue_t(1, 0)
    for t in range(1, NS):
        slot = (t - 1) & 1
        if t + 1 < NS:
            issue_t(t + 1, 1 - slot)
        tds[slot].wait()

        def radd(i, _):
            for u in range(2):
                j = (2 * i + u) * L
                part[pl.ds(j, L)] = (part[pl.ds(j, L)]
                                     + tbuf[pl.ds(slot * sl + j, L)])
            return _
        lax.fori_loop(0, sl // (2 * L), radd, None)
    pltpu.sync_copy(part, tabs_ref.at[pl.ds(c_id * tw + base, sl)])


def _apply_kernel_body(nv, hw, px_per_tile, blk, n, img_ref, seg_ref,
                       tabs_ref, fvr_ref, out_ref, repl, fvr, segb, xb, outb,
                       sem_a, sem_b, sem_oa, sem_ob):
    c_id = lax.axis_index("c")
    s_id = lax.axis_index("s")
    wid = c_id * NS + s_id
    tw = 4 * nv

    d0 = pltpu.async_copy(tabs_ref.at[pl.ds(0, tw)], repl, sem_a)
    dfv = pltpu.async_copy(fvr_ref, fvr, sem_b)
    d0.wait()
    csz = 3 * blk
    off = 0
    while off < tw:
        sz = min(csz, tw - off)
        pltpu.sync_copy(tabs_ref.at[pl.ds(tw + off, sz)],
                        xb.at[pl.ds(0, sz)])

        def tadd(i, _, off=off):
            j = i * L
            repl[pl.ds(off + j, L)] = (repl[pl.ds(off + j, L)]
                                       + xb[pl.ds(j, L)])
            return _
        lax.fori_loop(0, sz // L, tadd, None)
        off += sz
    dfv.wait()

    one = jnp.full((L,), 1.0, jnp.float32)

    def rbody(i, _):
        j = i * L
        cnt = repl[pl.ds(3 * nv + j, L)]
        cntm = jnp.maximum(cnt, one)
        for c in range(3):
            s = repl[pl.ds(c * nv + j, L)]
            f = fvr[pl.ds(c * nv + j, L)]
            repl[pl.ds(c * nv + j, L)] = f - s / cntm
        return _
    lax.fori_loop(0, nv // L, rbody, None)

    p0 = wid * px_per_tile
    b = p0 // hw
    i_base = p0 % hw
    nblk = px_per_tile // blk
    sems = (sem_a, sem_b)
    osems = (sem_oa, sem_ob)
    in_descs, out_descs = {}, {}

    def issue_in(bi, slot):
        off = bi * blk
        dd = [pltpu.async_copy(seg_ref.at[pl.ds(p0 + off, blk)],
                               segb.at[pl.ds(slot * blk, blk)], sems[slot])]
        for c in range(3):
            dd.append(pltpu.async_copy(
                img_ref.at[pl.ds((b * 3 + c) * hw + i_base + off, blk)],
                xb.at[pl.ds((slot * 3 + c) * blk, blk)], sems[slot]))
        in_descs[slot] = dd

    issue_in(0, 0)
    for bi in range(nblk):
        slot = bi & 1
        off = bi * blk
        if bi + 1 < nblk:
            issue_in(bi + 1, 1 - slot)
        if bi >= 2:
            for d in out_descs[slot]:
                d.wait()
        for d in in_descs[slot]:
            d.wait()

        def pvec(v):
            j0 = v * L
            poff = ((j0 >> 7) & 7) * 512 + (j0 >> 10) * 128 + (j0 & 127)
            sv = segb[pl.ds(slot * blk + poff, L)]
            obase = slot * (4 * blk) + (poff >> 7) * 512 + (poff & 127)
            for c in range(3):
                xv = xb[pl.ds((slot * 3 + c) * blk + j0, L)]
                rv = plsc.load_gather(repl, [sv + c * nv])
                outb[pl.ds(obase + c * 128, L)] = xv + rv

        def vec(i, _):
            pvec(2 * i)
            pvec(2 * i + 1)
            return _
        lax.fori_loop(0, blk // (2 * L), vec, None)
        out_descs[slot] = [
            pltpu.async_copy(outb.at[pl.ds(slot * (4 * blk), 4 * blk)],
                             out_ref.at[pl.ds((p0 + off) // 128 * 512,
                                              4 * blk)],
                             osems[slot])]
    for slot in (0, 1):
        if slot in out_descs:
            for d in out_descs[slot]:
                d.wait()


def kernel(img, fV_region, seg):
    B, C, H, W = img.shape
    nv = fV_region.shape[0]
    n = B * H * W
    hw = H * W
    px_per_tile = n // NW
    tw = 4 * nv

    img_flat = jnp.transpose(
        img.reshape(B, C, H // 8, 8, W // 128, 128),
        (0, 1, 2, 4, 3, 5)).reshape(-1)
    fvr_t = jnp.transpose(fV_region).reshape(-1)

    mesh = plsc.VectorSubcoreMesh(core_axis_name="c", subcore_axis_name="s")
    cparams = pltpu.CompilerParams(needs_layout_passes=False)

    blk_a = 4096
    sums = pl.kernel(
        functools.partial(_sums_kernel_body, nv, hw, px_per_tile, blk_a),
        out_type=jax.ShapeDtypeStruct((NC * tw,), jnp.float32),
        mesh=mesh,
        compiler_params=cparams,
        scratch_types=[
            pltpu.VMEM((tw,), jnp.float32),
            pltpu.VMEM((2 * blk_a,), jnp.int32),
            pltpu.VMEM((2 * 3 * blk_a,), jnp.float32),
            pltpu.VMEM_SHARED((NS * tw,), jnp.float32),
            pltpu.VMEM((tw // NS,), jnp.float32),
            pltpu.VMEM((2 * tw // NS,), jnp.float32),
            pltpu.SemaphoreType.DMA,
            pltpu.SemaphoreType.DMA,
        ],
    )(img_flat, seg)

    blk_c = 4096
    out = pl.kernel(
        functools.partial(_apply_kernel_body, nv, hw, px_per_tile, blk_c, n),
        out_type=jax.ShapeDtypeStruct((4 * n,), jnp.float32),
        mesh=mesh,
        compiler_params=cparams,
        scratch_types=[
            pltpu.VMEM((tw,), jnp.float32),
            pltpu.VMEM((3 * nv,), jnp.float32),
            pltpu.VMEM((2 * blk_c,), jnp.int32),
            pltpu.VMEM((2 * 3 * blk_c,), jnp.float32),
            pltpu.VMEM((2 * 4 * blk_c,), jnp.float32),
            pltpu.SemaphoreType.DMA,
            pltpu.SemaphoreType.DMA,
            pltpu.SemaphoreType.DMA,
            pltpu.SemaphoreType.DMA,
        ],
    )(img_flat, seg, sums, fvr_t)
    out4 = out.reshape(n // 128, 4, 128)[:, :3, :]
    return jnp.transpose(out4, (0, 2, 1)).reshape(n, 3)
